# Initial kernel scaffold; baseline (speedup 1.0000x reference)
import jax
import jax.numpy as jnp
from jax.experimental import pallas as pl


def kernel(x, edge_index, W0, W1, V0, V1, alphas):
    raise NotImplementedError("write your pallas kernel here")


# speedup vs baseline: 4.0466x; 4.0466x over previous
import functools

import jax
import jax.numpy as jnp
from jax import lax
from jax.experimental import pallas as pl
from jax.experimental.pallas import tpu as pltpu
from jax.experimental.pallas import tpu_sc as plsc

F32 = jnp.float32
NP = 10240
NW = 32
CH = 128
RB = 1280
D = 128


def _sc_agg_body(chunks, hw, srcr, dstr, agg_out, deg_out,
                 idx_s, idx_d, rows, deg, acc, sem):
    cid = lax.axis_index("c")
    sid = lax.axis_index("s")
    wid = sid * 2 + cid

    z16 = jnp.zeros((16,), F32)

    def zrow(i, c):
        for j in range(8):
            rows[i, pl.ds(j * 16, 16)] = z16
        return c
    lax.fori_loop(0, CH, zrow, 0)

    def zdeg(i, c):
        deg[pl.ds(i * 16, 16)] = z16
        return c
    lax.fori_loop(0, NP // 16, zdeg, 0)

    rows_per_tile = NP // 16
    base = sid * rows_per_tile
    for j in range(rows_per_tile // CH):
        pltpu.sync_copy(rows, acc.at[pl.ds(base + j * CH, CH)])
    plsc.subcore_barrier()

    pltpu.sync_copy(srcr.at[wid], idx_s)
    pltpu.sync_copy(dstr.at[wid], idx_d)

    ones16 = jnp.ones((16,), F32)

    def step(ci, c):
        pltpu.async_copy(hw.at[idx_s.at[ci]], rows, sem).wait()
        pltpu.sync_copy(rows, acc.at[idx_d.at[ci]], add=True)
        for j in range(8):
            dv = idx_d[ci, pl.ds(j * 16, 16)]
            plsc.addupdate_scatter(deg, [dv], ones16)
        return c
    lax.fori_loop(0, chunks, step, 0)

    plsc.subcore_barrier()
    pltpu.sync_copy(acc.at[pl.ds(base, rows_per_tile)],
                    agg_out.at[cid, pl.ds(base, rows_per_tile)])
    pltpu.sync_copy(deg, deg_out.at[wid])


def _make_sc_agg(chunks):
    mesh = plsc.VectorSubcoreMesh(core_axis_name="c", subcore_axis_name="s")
    return pl.kernel(
        functools.partial(_sc_agg_body, chunks),
        out_type=(jax.ShapeDtypeStruct((2, NP, D), F32),
                  jax.ShapeDtypeStruct((NW, NP), F32)),
        mesh=mesh,
        compiler_params=pltpu.CompilerParams(needs_layout_passes=False),
        scratch_types=[
            pltpu.VMEM((chunks, CH), jnp.int32),
            pltpu.VMEM((chunks, CH), jnp.int32),
            pltpu.VMEM((CH, D), F32),
            pltpu.VMEM((NP,), F32),
            pltpu.VMEM_SHARED((NP, D), F32),
            pltpu.SemaphoreType.DMA,
        ],
    )


def _mm_body(x_ref, w_ref, o_ref):
    o_ref[...] = jnp.dot(x_ref[...], w_ref[...], preferred_element_type=F32)


_mm = pl.pallas_call(
    _mm_body,
    grid=(NP // RB,),
    in_specs=[
        pl.BlockSpec((RB, D), lambda i: (i, 0)),
        pl.BlockSpec((D, D), lambda i: (0, 0)),
    ],
    out_specs=pl.BlockSpec((RB, D), lambda i: (i, 0)),
    out_shape=jax.ShapeDtypeStruct((NP, D), F32),
)


def _norm_mix(alpha_ref, aggp_ref, degp_ref, hprev_ref):
    agg = aggp_ref[0] + aggp_ref[1]
    dcol = lax.dot_general(degp_ref[...], jnp.ones((NW, 1), F32),
                           (((0,), (0,)), ((), ())),
                           preferred_element_type=F32)
    inv = 1.0 / jnp.maximum(dcol, 1.0)
    hn = jnp.maximum(agg * inv, 0.0)
    a = alpha_ref[0, 0]
    return a * hn + (1.0 - a) * hprev_ref[...]


def _stage_body(alpha_ref, aggp_ref, degp_ref, hprev_ref, w_ref, h_ref, hw_ref):
    h = _norm_mix(alpha_ref, aggp_ref, degp_ref, hprev_ref)
    h_ref[...] = h
    hw_ref[...] = jnp.dot(h, w_ref[...], preferred_element_type=F32)


_stage = pl.pallas_call(
    _stage_body,
    grid=(NP // RB,),
    in_specs=[
        pl.BlockSpec((1, 1), lambda i: (0, 0), memory_space=pltpu.SMEM),
        pl.BlockSpec((2, RB, D), lambda i: (0, i, 0)),
        pl.BlockSpec((NW, RB), lambda i: (0, i)),
        pl.BlockSpec((RB, D), lambda i: (i, 0)),
        pl.BlockSpec((D, D), lambda i: (0, 0)),
    ],
    out_specs=[
        pl.BlockSpec((RB, D), lambda i: (i, 0)),
        pl.BlockSpec((RB, D), lambda i: (i, 0)),
    ],
    out_shape=[
        jax.ShapeDtypeStruct((NP, D), F32),
        jax.ShapeDtypeStruct((NP, D), F32),
    ],
)


def _final_body(alpha_ref, aggp_ref, degp_ref, hprev_ref, xskip_ref, o_ref):
    h = _norm_mix(alpha_ref, aggp_ref, degp_ref, hprev_ref)
    o_ref[...] = jnp.concatenate([h, xskip_ref[...]], axis=1)


_final = pl.pallas_call(
    _final_body,
    grid=(NP // RB,),
    in_specs=[
        pl.BlockSpec((1, 1), lambda i: (0, 0), memory_space=pltpu.SMEM),
        pl.BlockSpec((2, RB, D), lambda i: (0, i, 0)),
        pl.BlockSpec((NW, RB), lambda i: (0, i)),
        pl.BlockSpec((RB, D), lambda i: (i, 0)),
        pl.BlockSpec((RB, D), lambda i: (i, 0)),
    ],
    out_specs=pl.BlockSpec((RB, 2 * D), lambda i: (i, 0)),
    out_shape=jax.ShapeDtypeStruct((NP, 2 * D), F32),
)


def _prep_edges(s, d, n_dummy):
    es = s.shape[0]
    chunks = -(-es // (NW * CH))
    tot = NW * chunks * CH
    sp = jnp.concatenate([s, jnp.zeros((tot - es,), jnp.int32)])
    dp = jnp.concatenate([d, jnp.full((tot - es,), n_dummy, jnp.int32)])
    return sp.reshape(NW, chunks, CH), dp.reshape(NW, chunks, CH), chunks


def kernel(x, edge_index, W0, W1, V0, V1, alphas):
    n, d = x.shape
    src = edge_index[0]
    dst = edge_index[1]

    srcf, dstf, chf = _prep_edges(src, dst, n)
    src2, dst2, ch2 = _prep_edges(src[::2], dst[::2], n)
    src4, dst4, ch4 = _prep_edges(src[::4], dst[::4], n)

    x_p = jnp.zeros((NP, d), F32).at[:n].set(x)

    sc_full = _make_sc_agg(chf)
    sc_s2 = _make_sc_agg(ch2)
    sc_s4 = _make_sc_agg(ch4)

    one = jnp.ones((1, 1), F32)
    a0 = alphas[0].reshape(1, 1)
    a1 = alphas[1].reshape(1, 1)

    hw0 = _mm(x_p, W0)
    agg1, degf = sc_full(hw0, srcf, dstf)
    h1, hw1 = _stage(one, agg1, degf, x_p, W1)
    agg2, _ = sc_full(hw1, srcf, dstf)
    h2, hw2 = _stage(one, agg2, degf, h1, V0)
    agg3, deg2 = sc_s2(hw2, src2, dst2)
    h3, hw3 = _stage(a0, agg3, deg2, h2, V1)
    agg4, deg4 = sc_s4(hw3, src4, dst4)
    out = _final(a1, agg4, deg4, h3, h2)
    return out[:n]
